# Initial kernel scaffold; baseline (speedup 1.0000x reference)
import jax
import jax.numpy as jnp
from jax.experimental import pallas as pl


def kernel(x, edge_index, batch_ids, atom_emb, Ws, bs, gammas, betas, W1, b1, W2, b2, W3, b3):
    raise NotImplementedError("write your pallas kernel here")


# speedup vs baseline: 5.5999x; 5.5999x over previous
import functools

import jax
import jax.numpy as jnp
from jax import lax
from jax.experimental import pallas as pl
from jax.experimental.pallas import tpu as pltpu
from jax.experimental.pallas import tpu_sc as plsc

N = 10000
E = 320000
H = 128
HH = 64
L = 4
G = 64
VOCAB = 128
NFEAT = 9

NSUB = 16
NTIL = 2 * NSUB
NROWS = 10240
RPT = NROWS // NSUB
CHUNK = 128
NCHT = 80
EPAD = NTIL * NCHT * CHUNK
ECH = EPAD // CHUNK


def _deg_body(dst_hbm, degp_hbm, dst_v, cnt_v):
    cid = lax.axis_index("c")
    sid = lax.axis_index("s")
    wid = cid * NSUB + sid
    pltpu.sync_copy(dst_hbm.at[pl.ds(wid * NCHT, NCHT)], dst_v)
    zeros16 = jnp.zeros((16,), jnp.float32)
    ones16 = jnp.ones((16,), jnp.float32)

    def zbody(i, carry):
        cnt_v[pl.ds(i * 16, 16)] = zeros16
        return carry

    lax.fori_loop(0, NROWS // 16, zbody, 0)

    def body(i, carry):
        j = i // 8
        k = i % 8
        idx = dst_v[j, pl.ds(k * 16, 16)]
        plsc.addupdate_scatter(cnt_v, [idx], ones16)
        return carry

    lax.fori_loop(0, NCHT * 8, body, 0)
    pltpu.sync_copy(cnt_v, degp_hbm.at[wid])


@functools.cache
def _deg_kernel():
    return pl.kernel(
        _deg_body,
        out_type=jax.ShapeDtypeStruct((NTIL, NROWS), jnp.float32),
        mesh=plsc.VectorSubcoreMesh(core_axis_name="c", subcore_axis_name="s"),
        scratch_types=[
            pltpu.VMEM((NCHT, CHUNK), jnp.int32),
            pltpu.VMEM((NROWS,), jnp.float32),
        ],
        compiler_params=pltpu.CompilerParams(needs_layout_passes=False),
    )


def _agg_body(t_hbm, src_hbm, dst_hbm, u2_hbm,
              src_v, dst_v, dstrow_v, rows_v, acc):
    cid = lax.axis_index("c")
    sid = lax.axis_index("s")
    wid = cid * NSUB + sid
    pltpu.sync_copy(src_hbm.at[pl.ds(wid * NCHT, NCHT)], src_v)
    pltpu.sync_copy(dst_hbm.at[pl.ds(wid * NCHT, NCHT)], dst_v)

    zeros16 = jnp.zeros((16,), jnp.float32)

    def zbody(i, carry):
        rows_v[i // 8, pl.ds((i % 8) * 16, 16)] = zeros16
        return carry

    lax.fori_loop(0, CHUNK * 8, zbody, 0)
    for r in range(RPT // CHUNK):
        pltpu.sync_copy(rows_v, acc.at[pl.ds(sid * RPT + r * CHUNK, CHUNK)])

    plsc.subcore_barrier()

    def body(j, carry):
        for k in range(CHUNK // 16):
            dstrow_v[pl.ds(k * 16, 16)] = dst_v[j, pl.ds(k * 16, 16)]
        pltpu.sync_copy(t_hbm.at[src_v.at[j]], rows_v)
        pltpu.sync_copy(rows_v, acc.at[dstrow_v], add=True)
        return carry

    lax.fori_loop(0, NCHT, body, 0)
    plsc.subcore_barrier()
    pltpu.sync_copy(acc.at[pl.ds(sid * RPT, RPT)],
                    u2_hbm.at[cid, pl.ds(sid * RPT, RPT)])


@functools.cache
def _agg_kernel():
    return pl.kernel(
        _agg_body,
        out_type=jax.ShapeDtypeStruct((2, NROWS, H), jnp.float32),
        mesh=plsc.VectorSubcoreMesh(core_axis_name="c", subcore_axis_name="s"),
        scratch_types=[
            pltpu.VMEM((NCHT, CHUNK), jnp.int32),
            pltpu.VMEM((NCHT, CHUNK), jnp.int32),
            pltpu.VMEM((CHUNK,), jnp.int32),
            pltpu.VMEM((CHUNK, H), jnp.float32),
            pltpu.VMEM_SHARED((NROWS, H), jnp.float32),
        ],
        compiler_params=pltpu.CompilerParams(needs_layout_passes=False),
    )


EMBBLK = 1024


def _emb_body(x_ref, emb_ref, h_ref):
    acc = jnp.zeros((EMBBLK, H), jnp.float32)
    iota = lax.broadcasted_iota(jnp.int32, (EMBBLK, VOCAB), 1)
    for f in range(NFEAT):
        xf = x_ref[:, f].reshape(EMBBLK, 1)
        onehot = jnp.where(xf == iota, 1.0, 0.0)
        acc = acc + jnp.dot(onehot, emb_ref[f],
                            preferred_element_type=jnp.float32, precision=lax.Precision.HIGHEST)
    h_ref[...] = acc


def _emb_call(x_p, atom_emb):
    return pl.pallas_call(
        _emb_body,
        grid=(NROWS // EMBBLK,),
        in_specs=[pl.BlockSpec((EMBBLK, NFEAT), lambda i: (i, 0)),
                  pl.BlockSpec((NFEAT, VOCAB, H), lambda i: (0, 0, 0))],
        out_specs=pl.BlockSpec((EMBBLK, H), lambda i: (i, 0)),
        out_shape=jax.ShapeDtypeStruct((NROWS, H), jnp.float32),
    )(x_p, atom_emb)


def _prep_body(h_ref, degp_ref, norm_ref, t_ref):
    ones_col = jnp.ones((NTIL, 1), jnp.float32)
    deg = lax.dot_general(degp_ref[...], ones_col, (((0,), (0,)), ((), ())),
                          preferred_element_type=jnp.float32, precision=lax.Precision.HIGHEST) + 1.0
    nrm2 = jnp.broadcast_to(lax.rsqrt(deg), (NROWS, H))
    norm_ref[...] = nrm2
    t_ref[...] = h_ref[...] * nrm2


def _layer_body(u2_ref, norm_ref, hprev_ref, w_ref, b_ref, g_ref,
                be_ref, h_ref, t_ref):
    nrm2 = norm_ref[...]
    u = u2_ref[0] + u2_ref[1] + hprev_ref[...] * nrm2
    s = u * nrm2
    v = jnp.dot(s, w_ref[...], preferred_element_type=jnp.float32, precision=lax.Precision.HIGHEST) + b_ref[...]
    rid = lax.broadcasted_iota(jnp.int32, (NROWS, 1), 0)
    mask = rid < N
    vm = jnp.where(mask, v, 0.0)
    mean = jnp.sum(vm, axis=0, keepdims=True) * (1.0 / N)
    dv = jnp.where(mask, v - mean, 0.0)
    var = jnp.sum(dv * dv, axis=0, keepdims=True) * (1.0 / N)
    hn = jnp.maximum((v - mean) * lax.rsqrt(var + 1e-5) * g_ref[...]
                     + be_ref[...], 0.0) + hprev_ref[...]
    hn = jnp.where(mask, hn, 0.0)
    h_ref[...] = hn
    t_ref[...] = hn * nrm2


def _head_body(h_ref, bid_ref, w1_ref, b1_ref, w2_ref, b2_ref, w3t_ref,
               b3_ref, out_ref):
    bid = bid_ref[...].reshape(NROWS, 1)
    gio = lax.broadcasted_iota(jnp.int32, (NROWS, G), 1)
    onehot = jnp.where(bid == gio, 1.0, 0.0)
    sums = lax.dot_general(onehot, h_ref[...], (((0,), (0,)), ((), ())),
                           preferred_element_type=jnp.float32, precision=lax.Precision.HIGHEST)
    ones_col = jnp.ones((NROWS, 1), jnp.float32)
    counts = lax.dot_general(onehot, ones_col, (((0,), (0,)), ((), ())),
                             preferred_element_type=jnp.float32, precision=lax.Precision.HIGHEST)
    pooled = sums / jnp.maximum(counts, 1.0)
    z = jnp.maximum(jnp.dot(pooled, w1_ref[...],
                            preferred_element_type=jnp.float32, precision=lax.Precision.HIGHEST)
                    + b1_ref[...], 0.0)
    z = jnp.maximum(jnp.dot(z, w2_ref[...],
                            preferred_element_type=jnp.float32, precision=lax.Precision.HIGHEST)
                    + b2_ref[...], 0.0)
    out_ref[...] = jnp.sum(z * w3t_ref[...], axis=1, keepdims=True) + b3_ref[...]


def _tc(body, out_shapes):
    return pl.pallas_call(body, out_shape=out_shapes)


def kernel(x, edge_index, batch_ids, atom_emb, Ws, bs, gammas, betas,
           W1, b1, W2, b2, W3, b3):
    f32, i32 = jnp.float32, jnp.int32
    x = x.astype(i32)
    src = edge_index[0].astype(i32)
    dst = edge_index[1].astype(i32)
    npad = NROWS - N
    epad = EPAD - E
    x_p = jnp.concatenate([x, jnp.zeros((npad, NFEAT), i32)], axis=0)
    src2d = jnp.concatenate([src, jnp.zeros((epad,), i32)]).reshape(ECH, CHUNK)
    dst2d = jnp.concatenate([dst, jnp.full((epad,), N, i32)]).reshape(ECH, CHUNK)
    bids_p = jnp.concatenate([batch_ids.astype(i32), jnp.full((npad,), G, i32)])

    h = _emb_call(x_p, atom_emb)
    degp = _deg_kernel()(dst2d)
    norm2, t = _tc(_prep_body, [jax.ShapeDtypeStruct((NROWS, H), f32),
                                jax.ShapeDtypeStruct((NROWS, H), f32)])(h, degp)
    layer_out = [jax.ShapeDtypeStruct((NROWS, H), f32),
                 jax.ShapeDtypeStruct((NROWS, H), f32)]
    for i in range(L):
        u2 = _agg_kernel()(t, src2d, dst2d)
        h, t = _tc(_layer_body, layer_out)(
            u2, norm2, h, Ws[i], bs[i].reshape(1, H),
            gammas[i].reshape(1, H), betas[i].reshape(1, H))
    out = _tc(_head_body, jax.ShapeDtypeStruct((G, 1), f32))(
        h, bids_p, W1, b1.reshape(1, HH), W2, b2.reshape(1, H // 4),
        W3.reshape(1, H // 4), b3.reshape(1, 1))
    return out


# speedup vs baseline: 5.6049x; 1.0009x over previous
import functools

import jax
import jax.numpy as jnp
from jax import lax
from jax.experimental import pallas as pl
from jax.experimental.pallas import tpu as pltpu
from jax.experimental.pallas import tpu_sc as plsc

N = 10000
E = 320000
H = 128
HH = 64
L = 4
G = 64
VOCAB = 128
NFEAT = 9

NSUB = 16
NTIL = 2 * NSUB
NROWS = 10240
RPT = NROWS // NSUB
CHUNK = 128
NCHT = 80
EPAD = NTIL * NCHT * CHUNK
ECH = EPAD // CHUNK


def _deg_body(dst_hbm, degp_hbm, dst_v, cnt_v):
    cid = lax.axis_index("c")
    sid = lax.axis_index("s")
    wid = cid * NSUB + sid
    pltpu.sync_copy(dst_hbm.at[pl.ds(wid * NCHT, NCHT)], dst_v)
    zeros16 = jnp.zeros((16,), jnp.float32)
    ones16 = jnp.ones((16,), jnp.float32)

    def zbody(i, carry):
        cnt_v[pl.ds(i * 16, 16)] = zeros16
        return carry

    lax.fori_loop(0, NROWS // 16, zbody, 0)

    def body(i, carry):
        j = i // 8
        k = i % 8
        idx = dst_v[j, pl.ds(k * 16, 16)]
        plsc.addupdate_scatter(cnt_v, [idx], ones16)
        return carry

    lax.fori_loop(0, NCHT * 8, body, 0)
    pltpu.sync_copy(cnt_v, degp_hbm.at[wid])


@functools.cache
def _deg_kernel():
    return pl.kernel(
        _deg_body,
        out_type=jax.ShapeDtypeStruct((NTIL, NROWS), jnp.float32),
        mesh=plsc.VectorSubcoreMesh(core_axis_name="c", subcore_axis_name="s"),
        scratch_types=[
            pltpu.VMEM((NCHT, CHUNK), jnp.int32),
            pltpu.VMEM((NROWS,), jnp.float32),
        ],
        compiler_params=pltpu.CompilerParams(needs_layout_passes=False),
    )


def _agg_body(t_hbm, src_hbm, dst_hbm, u2_hbm,
              src_v, dst_v, dstrow_v, rows_v, acc):
    cid = lax.axis_index("c")
    sid = lax.axis_index("s")
    wid = cid * NSUB + sid
    pltpu.sync_copy(src_hbm.at[pl.ds(wid * NCHT, NCHT)], src_v)
    pltpu.sync_copy(dst_hbm.at[pl.ds(wid * NCHT, NCHT)], dst_v)

    zeros16 = jnp.zeros((16,), jnp.float32)

    def zbody(i, carry):
        rows_v[i // 8, pl.ds((i % 8) * 16, 16)] = zeros16
        return carry

    lax.fori_loop(0, CHUNK * 8, zbody, 0)
    for r in range(RPT // CHUNK):
        pltpu.sync_copy(rows_v, acc.at[pl.ds(sid * RPT + r * CHUNK, CHUNK)])

    plsc.subcore_barrier()

    def body(j, carry):
        for k in range(CHUNK // 16):
            dstrow_v[pl.ds(k * 16, 16)] = dst_v[j, pl.ds(k * 16, 16)]
        pltpu.sync_copy(t_hbm.at[src_v.at[j]], rows_v)
        pltpu.sync_copy(rows_v, acc.at[dstrow_v], add=True)
        return carry

    lax.fori_loop(0, NCHT, body, 0)
    plsc.subcore_barrier()
    pltpu.sync_copy(acc.at[pl.ds(sid * RPT, RPT)],
                    u2_hbm.at[cid, pl.ds(sid * RPT, RPT)])


@functools.cache
def _agg_kernel():
    return pl.kernel(
        _agg_body,
        out_type=jax.ShapeDtypeStruct((2, NROWS, H), jnp.float32),
        mesh=plsc.VectorSubcoreMesh(core_axis_name="c", subcore_axis_name="s"),
        scratch_types=[
            pltpu.VMEM((NCHT, CHUNK), jnp.int32),
            pltpu.VMEM((NCHT, CHUNK), jnp.int32),
            pltpu.VMEM((CHUNK,), jnp.int32),
            pltpu.VMEM((CHUNK, H), jnp.float32),
            pltpu.VMEM_SHARED((NROWS, H), jnp.float32),
        ],
        compiler_params=pltpu.CompilerParams(needs_layout_passes=False),
    )


EMBBLK = 1024


def _emb_body(x_ref, emb_ref, h_ref):
    acc = jnp.zeros((EMBBLK, H), jnp.float32)
    iota = lax.broadcasted_iota(jnp.int32, (EMBBLK, VOCAB), 1)
    for f in range(NFEAT):
        xf = x_ref[:, f].reshape(EMBBLK, 1)
        onehot = jnp.where(xf == iota, 1.0, 0.0)
        acc = acc + jnp.dot(onehot, emb_ref[f],
                            preferred_element_type=jnp.float32, precision=lax.Precision.HIGHEST)
    h_ref[...] = acc


def _emb_call(x_p, atom_emb):
    return pl.pallas_call(
        _emb_body,
        grid=(NROWS // EMBBLK,),
        in_specs=[pl.BlockSpec((EMBBLK, NFEAT), lambda i: (i, 0)),
                  pl.BlockSpec((NFEAT, VOCAB, H), lambda i: (0, 0, 0))],
        out_specs=pl.BlockSpec((EMBBLK, H), lambda i: (i, 0)),
        out_shape=jax.ShapeDtypeStruct((NROWS, H), jnp.float32),
    )(x_p, atom_emb)


def _prep_body(h_ref, degp_ref, norm_ref, t_ref):
    ones_col = jnp.ones((NTIL, 1), jnp.float32)
    deg = lax.dot_general(degp_ref[...], ones_col, (((0,), (0,)), ((), ())),
                          preferred_element_type=jnp.float32, precision=lax.Precision.HIGHEST) + 1.0
    nrm2 = jnp.broadcast_to(lax.rsqrt(deg), (NROWS, H))
    norm_ref[...] = nrm2
    t_ref[...] = h_ref[...] * nrm2


def _layer_body(u2_ref, norm_ref, hprev_ref, w_ref, b_ref, g_ref,
                be_ref, h_ref, t_ref):
    nrm2 = norm_ref[...]
    u = u2_ref[0] + u2_ref[1] + hprev_ref[...] * nrm2
    s = u * nrm2
    v = jnp.dot(s, w_ref[...], preferred_element_type=jnp.float32, precision=lax.Precision.HIGHEST) + b_ref[...]
    rid = lax.broadcasted_iota(jnp.int32, (NROWS, 1), 0)
    mask = rid < N
    vm = jnp.where(mask, v, 0.0)
    mean = jnp.sum(vm, axis=0, keepdims=True) * (1.0 / N)
    dv = jnp.where(mask, v - mean, 0.0)
    var = jnp.sum(dv * dv, axis=0, keepdims=True) * (1.0 / N)
    hn = jnp.maximum((v - mean) * lax.rsqrt(var + 1e-5) * g_ref[...]
                     + be_ref[...], 0.0) + hprev_ref[...]
    hn = jnp.where(mask, hn, 0.0)
    h_ref[...] = hn
    t_ref[...] = hn * nrm2


def _head_body(h_ref, bid_ref, w1_ref, b1_ref, w2_ref, b2_ref, w3t_ref,
               b3_ref, out_ref):
    bid = bid_ref[...].reshape(NROWS, 1)
    gio = lax.broadcasted_iota(jnp.int32, (NROWS, G), 1)
    onehot = jnp.where(bid == gio, 1.0, 0.0)
    sums = lax.dot_general(onehot, h_ref[...], (((0,), (0,)), ((), ())),
                           preferred_element_type=jnp.float32, precision=lax.Precision.HIGHEST)
    ones_col = jnp.ones((NROWS, 1), jnp.float32)
    counts = lax.dot_general(onehot, ones_col, (((0,), (0,)), ((), ())),
                             preferred_element_type=jnp.float32, precision=lax.Precision.HIGHEST)
    pooled = sums / jnp.maximum(counts, 1.0)
    z = jnp.maximum(jnp.dot(pooled, w1_ref[...],
                            preferred_element_type=jnp.float32, precision=lax.Precision.HIGHEST)
                    + b1_ref[...], 0.0)
    z = jnp.maximum(jnp.dot(z, w2_ref[...],
                            preferred_element_type=jnp.float32, precision=lax.Precision.HIGHEST)
                    + b2_ref[...], 0.0)
    out_ref[...] = jnp.sum(z * w3t_ref[...], axis=1, keepdims=True) + b3_ref[...]


def _tc(body, out_shapes):
    return pl.pallas_call(body, out_shape=out_shapes)


def kernel(x, edge_index, batch_ids, atom_emb, Ws, bs, gammas, betas,
           W1, b1, W2, b2, W3, b3):
    f32, i32 = jnp.float32, jnp.int32
    x = x.astype(i32)
    src = edge_index[0].astype(i32)
    dst = edge_index[1].astype(i32)
    npad = NROWS - N
    epad = EPAD - E
    x_p = jnp.concatenate([x, jnp.zeros((npad, NFEAT), i32)], axis=0)
    pad_dst = N + jnp.arange(epad, dtype=i32) % jnp.int32(npad)
    src2d = jnp.concatenate([src, jnp.zeros((epad,), i32)]).reshape(ECH, CHUNK)
    dst2d = jnp.concatenate([dst, pad_dst]).reshape(ECH, CHUNK)
    bids_p = jnp.concatenate([batch_ids.astype(i32), jnp.full((npad,), G, i32)])

    h = _emb_call(x_p, atom_emb)
    degp = _deg_kernel()(dst2d)
    norm2, t = _tc(_prep_body, [jax.ShapeDtypeStruct((NROWS, H), f32),
                                jax.ShapeDtypeStruct((NROWS, H), f32)])(h, degp)
    layer_out = [jax.ShapeDtypeStruct((NROWS, H), f32),
                 jax.ShapeDtypeStruct((NROWS, H), f32)]
    for i in range(L):
        u2 = _agg_kernel()(t, src2d, dst2d)
        h, t = _tc(_layer_body, layer_out)(
            u2, norm2, h, Ws[i], bs[i].reshape(1, H),
            gammas[i].reshape(1, H), betas[i].reshape(1, H))
    out = _tc(_head_body, jax.ShapeDtypeStruct((G, 1), f32))(
        h, bids_p, W1, b1.reshape(1, HH), W2, b2.reshape(1, H // 4),
        W3.reshape(1, H // 4), b3.reshape(1, 1))
    return out
